# baseline (speedup 1.0000x reference)
import jax
import jax.numpy as jnp
from jax import lax
from jax.experimental import pallas as pl
from jax.experimental.pallas import tpu as pltpu, tpu_sc as plsc

_N = 10000
_NPAD = 10240
_G = 64
_D = 256
_CH = 64
_KMAX = (_N + 1) // 2 + _CH


def _tc1_body(x_ref, brow_ref, w_ref, fk_ref, gate_ref, counts_ref,
              starts_ref, inv_ref):
    x = x_ref[...]
    brow = brow_ref[...]
    w_row = w_ref[...]

    f32 = jnp.float32
    dg = jax.lax.dot_general
    wn = jax.lax.rsqrt(jnp.sum(w_row * w_row))
    score = dg(w_row, x, (((1,), (1,)), ((), ())),
               preferred_element_type=f32) * wn

    sc = jnp.where(score == 0.0, 0.0, score)
    sbits = jax.lax.bitcast_convert_type(sc, jnp.int32)
    key_asc = jnp.where(sbits >= 0, sbits, sbits ^ jnp.int32(0x7FFFFFFF))
    fk = -key_asc

    gids_col = jax.lax.broadcasted_iota(jnp.int32, (_G, 1), 0)
    gids_row = jax.lax.broadcasted_iota(jnp.int32, (1, _G), 1)
    ohT = jnp.where(gids_col == brow, 1.0, 0.0).astype(f32)
    ones_row = jnp.ones((1, _NPAD), f32)
    counts_f = dg(ones_row, ohT, (((1,), (1,)), ((), ())),
                  preferred_element_type=f32)
    slt = jnp.where(gids_col < gids_row, 1.0, 0.0).astype(f32)
    starts_f = dg(counts_f, slt, (((1,), (0,)), ((), ())),
                  preferred_element_type=f32)

    fk_ref[...] = fk
    gate_ref[...] = jnp.tanh(score)
    counts_ref[...] = counts_f.astype(jnp.int32)
    starts_ref[...] = starts_f.astype(jnp.int32)
    inv_ref[...] = 1.0 / jnp.maximum(jnp.ceil(0.5 * counts_f), 1.0)


def _splat(v):
    return jnp.full((16,), v, jnp.int32)


def _scal(vec):
    return lax.reduce_max(vec, axes=(0,))


def _sc_body(x_hbm, fk_hbm, gate_hbm, counts_hbm, starts_hbm, invk_hbm,
             out_hbm, fk_v, gate_v, counts_v, starts_v, invk_v, idx_v,
             idxc_v, rows_v, acc_v, sem):
    wid = lax.axis_index("s") * 2 + lax.axis_index("c")
    pltpu.sync_copy(fk_hbm, fk_v)
    pltpu.sync_copy(gate_hbm, gate_v)
    pltpu.sync_copy(counts_hbm, counts_v)
    pltpu.sync_copy(starts_hbm, starts_v)
    pltpu.sync_copy(invk_hbm, invk_v)
    lanes = lax.iota(jnp.int32, 16)
    lanesf = lanes.astype(jnp.float32)
    imax = jnp.int32(2147483647)
    imin = jnp.int32(-2147483647 - 1)
    zero16 = jnp.zeros((16,), jnp.float32)

    for gi in range(2):
        g = wid * 2 + gi
        gv = _splat(g)
        n = _scal(plsc.load_gather(counts_v, [gv]))
        s = _scal(plsc.load_gather(starts_v, [gv]))
        k = (n + 1) // 2
        nchunk = (n + 15) // 16

        def count_le(t):
            def body(i, acc):
                idx = s + i * 16 + lanes
                valid = (i * 16 + lanes) < n
                v = plsc.load_gather(fk_v, [idx], mask=valid)
                c = jnp.where(valid & (v <= t), 1, 0)
                return acc + lax.reduce_sum(c, axes=(0,))
            return lax.fori_loop(0, nchunk, body, jnp.int32(0))

        def mm_body(i, st):
            lo, hi = st
            idx = s + i * 16 + lanes
            valid = (i * 16 + lanes) < n
            v = plsc.load_gather(fk_v, [idx], mask=valid)
            lo = jnp.minimum(lo, lax.reduce_min(
                jnp.where(valid, v, imax), axes=(0,)))
            hi = jnp.maximum(hi, lax.reduce_max(
                jnp.where(valid, v, imin), axes=(0,)))
            return lo, hi

        lo, hi = lax.fori_loop(0, nchunk, mm_body, (imax, imin))
        lo = jnp.minimum(lo, hi)

        def bs_cond(st):
            a, b = st
            return a < b

        def bs_body(st):
            a, b = st
            mid = (a >> 1) + (b >> 1) + (a & b & 1)
            c = count_le(mid)
            return (jnp.where(c >= k, a, mid + 1),
                    jnp.where(c >= k, mid, b))

        vfk, _ = lax.while_loop(bs_cond, bs_body, (lo, hi))
        nless = count_le(vfk - 1)
        need = k - nless

        def eq_body(i, st):
            cnt, best = st
            idx = s + i * 16 + lanes
            valid = (i * 16 + lanes) < n
            v = plsc.load_gather(fk_v, [idx], mask=valid)
            eq = valid & (v == vfk)
            pref = plsc.cumsum(jnp.where(eq, 1, 0)) + cnt
            hit = eq & (pref == need)
            best = jnp.maximum(best, lax.reduce_max(
                jnp.where(hit, idx, jnp.int32(-1)), axes=(0,)))
            return _scal(pref), best

        _, vidx = lax.fori_loop(0, nchunk, eq_body, (jnp.int32(0),
                                                     jnp.int32(-1)))

        def compact_body(i, pos):
            idx = s + i * 16 + lanes
            valid = (i * 16 + lanes) < n
            v = plsc.load_gather(fk_v, [idx], mask=valid)
            sel = valid & ((v < vfk) | ((v == vfk) & (idx <= vidx)))
            pref = plsc.cumsum(jnp.where(sel, 1, 0))
            plsc.store_scatter(idx_v, [pos + pref - 1], idx, mask=sel)
            return pos + _scal(pref)

        ksel = lax.fori_loop(0, nchunk, compact_body, jnp.int32(0))
        for j in range(_CH // 16):
            plsc.store_scatter(idx_v, [ksel + j * 16 + lanes],
                               _splat(0))

        nch = (ksel + _CH - 1) // _CH

        def chunk_body(c, acc):
            base = c * _CH
            for t in range(_CH // 16):
                idxc_v[pl.ds(t * 16, 16)] = idx_v[pl.ds(base + t * 16, 16)]
            pltpu.async_copy(x_hbm.at[idxc_v], rows_v, sem).wait()
            m = jnp.minimum(ksel - base, _CH)

            def row_body(r, acc2):
                q = r // 16
                gates16 = plsc.load_gather(
                    gate_v, [plsc.load_gather(idxc_v, [q * 16 + lanes])])
                ga = lax.reduce_max(
                    jnp.where(lanes == (r - q * 16), gates16,
                              jnp.float32(-2.0)), axes=(0,))
                return tuple(
                    acc2[j] + ga * rows_v[r, pl.ds(j * 16, 16)]
                    for j in range(_D // 16))

            return lax.fori_loop(0, m, row_body, acc)

        acc0 = tuple(zero16 for _ in range(_D // 16))
        acc = lax.fori_loop(0, nch, chunk_body, acc0)

        inv = lax.reduce_max(plsc.load_gather(invk_v, [gv]), axes=(0,))
        for j in range(_D // 16):
            acc_v[pl.ds(j * 16, 16)] = acc[j] * inv
        pltpu.sync_copy(acc_v, out_hbm.at[g])


def _make_sc():
    mesh = plsc.VectorSubcoreMesh(core_axis_name="c", subcore_axis_name="s")
    return pl.kernel(
        _sc_body,
        out_type=jax.ShapeDtypeStruct((_G, _D), jnp.float32),
        mesh=mesh,
        compiler_params=pltpu.CompilerParams(needs_layout_passes=False),
        scratch_types=[
            pltpu.VMEM((_NPAD,), jnp.int32),
            pltpu.VMEM((_NPAD,), jnp.float32),
            pltpu.VMEM((_G,), jnp.int32),
            pltpu.VMEM((_G,), jnp.int32),
            pltpu.VMEM((_G,), jnp.float32),
            pltpu.VMEM((_KMAX,), jnp.int32),
            pltpu.VMEM((_CH,), jnp.int32),
            pltpu.VMEM((_CH, _D), jnp.float32),
            pltpu.VMEM((_D,), jnp.float32),
            pltpu.SemaphoreType.DMA,
        ],
    )


def kernel(x, edge_index, batch, w):
    del edge_index
    f32 = jnp.float32
    xp = jnp.zeros((_NPAD, _D), f32).at[:_N].set(x)
    brow = jnp.full((1, _NPAD), _G, jnp.int32).at[0, :_N].set(batch)

    fk, gate, counts, starts, invk = pl.pallas_call(
        _tc1_body,
        out_shape=[
            jax.ShapeDtypeStruct((1, _NPAD), jnp.int32),
            jax.ShapeDtypeStruct((1, _NPAD), f32),
            jax.ShapeDtypeStruct((1, _G), jnp.int32),
            jax.ShapeDtypeStruct((1, _G), jnp.int32),
            jax.ShapeDtypeStruct((1, _G), f32),
        ],
    )(xp, brow, w.reshape(1, _D))

    out = _make_sc()(xp, fk.reshape(_NPAD), gate.reshape(_NPAD),
                     counts.reshape(_G), starts.reshape(_G),
                     invk.reshape(_G))
    return out
